# baseline (speedup 1.0000x reference)
import functools

import jax
import jax.numpy as jnp
from jax import lax
from jax.experimental import pallas as pl
from jax.experimental.pallas import tpu as pltpu
from jax.experimental.pallas import tpu_sc as plsc

_VOCAB = 1000
_VPAD = 1024
_NSLICE = _VPAD // 128
_BATCH = 16384
_NC = 2
_NS = 16
_NW = _NC * _NS
_BPW = _BATCH // _NW
_CB = 128
_NCB = _BPW // _CB

_mesh = plsc.VectorSubcoreMesh(core_axis_name="c", subcore_axis_name="s")


@functools.partial(
    pl.kernel,
    out_type=jax.ShapeDtypeStruct((_VOCAB, _BATCH), jnp.float32),
    mesh=_mesh,
    scratch_types=[
        pltpu.VMEM((_CB,), jnp.int32),
        pltpu.VMEM((4 * _CB,), jnp.int32),
        pltpu.VMEM((4 * _CB, 128), jnp.float32),
        pltpu.VMEM((256, 128), jnp.float32),
        pltpu.SemaphoreType.DMA,
    ],
    compiler_params=pltpu.CompilerParams(needs_layout_passes=False),
)
def _tgather_kernel(tab_hbm, idx_hbm, out_hbm, idx_c, gidx, buf_a, buf_t, sem):
    wid = lax.axis_index("s") * _NC + lax.axis_index("c")
    base = wid * _BPW
    iota16 = lax.iota(jnp.int32, 16)

    for cb in range(_NCB):
        b0 = pl.multiple_of(base + cb * _CB, _CB)
        pltpu.sync_copy(idx_hbm.at[pl.ds(b0, _CB)], idx_c)
        for p in range(2):
            for jl in range(4):
                for b16 in range(_CB // 16):
                    vec = idx_c[pl.ds(b16 * 16, 16)] * 8 + (4 * p + jl)
                    gidx[pl.ds(jl * _CB + b16 * 16, 16)] = vec
            pltpu.async_copy(tab_hbm.at[gidx], buf_a, sem).wait()
            for sub in range(2):
                v0 = 512 * p + 256 * sub
                vs = min(256, _VOCAB - v0)

                def body(r, carry, sub=sub):
                    jl = 2 * sub + (r >> 7)
                    col = lax.full((16,), r & 127, jnp.int32)
                    for b16 in range(_CB // 16):
                        rows = iota16 + (jl * _CB + b16 * 16)
                        x = plsc.load_gather(buf_a, [rows, col])
                        buf_t[r, pl.ds(b16 * 16, 16)] = x
                    return carry

                lax.fori_loop(0, vs, body, 0)
                pltpu.sync_copy(
                    buf_t.at[pl.ds(0, vs)],
                    out_hbm.at[pl.ds(v0, vs), pl.ds(b0, _CB)],
                )


def kernel(idx, embedding):
    tab = jnp.pad(embedding, ((0, 0), (0, _VPAD - _VOCAB)))
    tab = tab.reshape(_VOCAB * _NSLICE, 128)
    return _tgather_kernel(tab, idx).T


# speedup vs baseline: 1.6520x; 1.6520x over previous
import functools

import jax
import jax.numpy as jnp
from jax import lax
from jax.experimental import pallas as pl
from jax.experimental.pallas import tpu as pltpu
from jax.experimental.pallas import tpu_sc as plsc

_VOCAB = 1000
_VPAD = 1024
_NSLICE = _VPAD // 128
_BATCH = 16384
_NC = 2
_NS = 16
_NW = _NC * _NS
_BPW = _BATCH // _NW
_CB = 128
_NCB = _BPW // _CB

_mesh = plsc.VectorSubcoreMesh(core_axis_name="c", subcore_axis_name="s")


@functools.partial(
    pl.kernel,
    out_type=jax.ShapeDtypeStruct((_VOCAB, _BATCH), jnp.float32),
    mesh=_mesh,
    scratch_types=[
        pltpu.VMEM((_CB,), jnp.int32),
        pltpu.VMEM((4 * _CB,), jnp.int32),
        pltpu.VMEM((4 * _CB, 128), jnp.float32),
        pltpu.VMEM((128, 128), jnp.float32),
        pltpu.SemaphoreType.DMA,
    ],
    compiler_params=pltpu.CompilerParams(needs_layout_passes=False),
)
def _tgather_kernel(tab_hbm, idx_hbm, out_hbm, idx_c, gidx, buf_a, buf_t, sem):
    wid = lax.axis_index("s") * _NC + lax.axis_index("c")
    base = wid * _BPW
    iota16 = lax.iota(jnp.int32, 16)

    def chunk_body(cb, carry):
        b0 = pl.multiple_of(base + cb * _CB, _CB)
        pltpu.sync_copy(idx_hbm.at[pl.ds(b0, _CB)], idx_c)
        for p in range(2):
            for jl in range(4):
                for b16 in range(_CB // 16):
                    vec = idx_c[pl.ds(b16 * 16, 16)] * 8 + (4 * p + jl)
                    gidx[pl.ds(jl * _CB + b16 * 16, 16)] = vec
            pltpu.async_copy(tab_hbm.at[gidx], buf_a, sem).wait()
            for vb in range(4):
                v0 = 512 * p + 128 * vb
                vs = min(128, _VOCAB - v0)
                rows = [iota16 + (vb * _CB + b16 * 16) for b16 in range(_CB // 16)]

                @plsc.parallel_loop(0, vs, unroll=4)
                def body(r):
                    col = lax.full((16,), r, jnp.int32)
                    for b16 in range(_CB // 16):
                        x = plsc.load_gather(buf_a, [rows[b16], col])
                        buf_t[r, pl.ds(b16 * 16, 16)] = x

                pltpu.sync_copy(
                    buf_t.at[pl.ds(0, vs)],
                    out_hbm.at[pl.ds(v0, vs), pl.ds(b0, _CB)],
                )
        return carry

    lax.fori_loop(0, _NCB, chunk_body, 0)


def kernel(idx, embedding):
    tab = jnp.pad(embedding, ((0, 0), (0, _VPAD - _VOCAB)))
    tab = tab.reshape(_VOCAB * _NSLICE, 128)
    return _tgather_kernel(tab, idx).T


# speedup vs baseline: 4.6368x; 2.8068x over previous
import functools

import jax
import jax.numpy as jnp
from jax import lax
from jax.experimental import pallas as pl
from jax.experimental.pallas import tpu as pltpu
from jax.experimental.pallas import tpu_sc as plsc

_VOCAB = 1000
_VPAD = 1024
_NSLICE = _VPAD // 128
_BATCH = 16384
_NC = 2
_NS = 16
_NW = _NC * _NS
_VPW = 32
_NOCT = 4
_BC = 2048
_NBC = _BATCH // _BC

_mesh = plsc.VectorSubcoreMesh(core_axis_name="c", subcore_axis_name="s")


@functools.partial(
    pl.kernel,
    out_type=jax.ShapeDtypeStruct((_VOCAB, _BATCH), jnp.float32),
    mesh=_mesh,
    scratch_types=[
        pltpu.VMEM((_BATCH,), jnp.int32),
        pltpu.VMEM((_VPW * _NSLICE, 128), jnp.float32),
        pltpu.VMEM((_VPW, _BC), jnp.float32),
    ],
    compiler_params=pltpu.CompilerParams(needs_layout_passes=False),
)
def _tgather_kernel(tab_hbm, idx_hbm, out_hbm, idx_l, tab_l, buf):
    wid = lax.axis_index("s") * _NC + lax.axis_index("c")
    v0 = wid * _VPW
    pltpu.sync_copy(idx_hbm, idx_l)
    for k in range(_NOCT):
        @pl.when(v0 + k * 8 < _VOCAB)
        def _load(k=k):
            pltpu.sync_copy(
                tab_hbm.at[pl.ds((v0 + k * 8) * _NSLICE, 8 * _NSLICE)],
                tab_l.at[pl.ds(k * 8 * _NSLICE, 8 * _NSLICE)],
            )

    def chunk_body(bc, carry):
        b0 = pl.multiple_of(bc * _BC, _BC)

        @plsc.parallel_loop(0, _BC // 16, unroll=2)
        def body(b16):
            idxv = idx_l[pl.ds(b0 + b16 * 16, 16)]
            rowoff = lax.shift_right_logical(idxv, 7)
            col = lax.bitwise_and(idxv, 127)
            for vl in range(_VPW):
                rows = rowoff + vl * _NSLICE
                x = plsc.load_gather(tab_l, [rows, col])
                buf[vl, pl.ds(b16 * 16, 16)] = x

        for k in range(_NOCT):
            @pl.when(v0 + k * 8 < _VOCAB)
            def _store(k=k):
                pltpu.sync_copy(
                    buf.at[pl.ds(k * 8, 8)],
                    out_hbm.at[pl.ds(v0 + k * 8, 8), pl.ds(b0, _BC)],
                )
        return carry

    lax.fori_loop(0, _NBC, chunk_body, 0)


def kernel(idx, embedding):
    tab = jnp.pad(embedding.T, ((0, 0), (0, _VPAD - _VOCAB)))
    tab = tab.reshape(_VOCAB * _NSLICE, 128)
    return _tgather_kernel(tab, idx).T


# speedup vs baseline: 6.0249x; 1.2994x over previous
import functools

import jax
import jax.numpy as jnp
from jax import lax
from jax.experimental import pallas as pl
from jax.experimental.pallas import tpu as pltpu
from jax.experimental.pallas import tpu_sc as plsc

_VOCAB = 1000
_VPAD = 1024
_BATCH = 16384
_NC = 2
_NS = 16
_NW = _NC * _NS
_VPW = 32
_NOCT = 4
_BC = 1024
_NBC = _BATCH // _BC

_mesh = plsc.VectorSubcoreMesh(core_axis_name="c", subcore_axis_name="s")


@functools.partial(
    pl.kernel,
    out_type=jax.ShapeDtypeStruct((_VOCAB, _BATCH), jnp.float32),
    mesh=_mesh,
    scratch_types=[
        pltpu.VMEM((_BATCH,), jnp.int32),
        pltpu.VMEM((_VPW * _VPAD,), jnp.float32),
        pltpu.VMEM((2, _VPW, _BC), jnp.float32),
        pltpu.SemaphoreType.DMA,
        pltpu.SemaphoreType.DMA,
    ],
    compiler_params=pltpu.CompilerParams(needs_layout_passes=False),
)
def _tgather_kernel(tab_hbm, idx_hbm, out_hbm, idx_l, tab_l, bufs, sem0, sem1):
    wid = lax.axis_index("s") * _NC + lax.axis_index("c")
    v0 = wid * _VPW
    sems = (sem0, sem1)
    pltpu.sync_copy(idx_hbm, idx_l)
    for k in range(_NOCT):
        @pl.when(v0 + k * 8 < _VOCAB)
        def _load(k=k):
            pltpu.sync_copy(
                tab_hbm.at[pl.ds((v0 + k * 8) * _VPAD, 8 * _VPAD)],
                tab_l.at[pl.ds(k * 8 * _VPAD, 8 * _VPAD)],
            )

    def store_descs(par, b0):
        descs = []
        for k in range(_NOCT):
            descs.append((
                v0 + k * 8 < _VOCAB,
                bufs.at[par, pl.ds(k * 8, 8)],
                out_hbm.at[pl.ds(v0 + k * 8, 8), pl.ds(b0, _BC)],
                sems[par],
            ))
        return descs

    def do_chunk(bc2, par, drain):
        b0 = pl.multiple_of((2 * bc2 + par) * _BC, _BC)

        @pl.when(drain)
        def _drain():
            for cond, src, dst, sem in store_descs(par, b0):
                @pl.when(cond)
                def _w(src=src, dst=dst, sem=sem):
                    pltpu.make_async_copy(src, dst, sem).wait()

        @plsc.parallel_loop(0, _BC // 16, unroll=2)
        def body(b16):
            idxv = idx_l[pl.ds(b0 + b16 * 16, 16)]
            for vl in range(_VPW):
                x = plsc.load_gather(tab_l, [idxv + vl * _VPAD])
                bufs[par, vl, pl.ds(b16 * 16, 16)] = x

        for cond, src, dst, sem in store_descs(par, b0):
            @pl.when(cond)
            def _s(src=src, dst=dst, sem=sem):
                pltpu.async_copy(src, dst, sem)

    def chunk_body(bc2, carry):
        do_chunk(bc2, 0, bc2 >= 1)
        do_chunk(bc2, 1, bc2 >= 1)
        return carry

    lax.fori_loop(0, _NBC // 2, chunk_body, 0)

    for par in (0, 1):
        b0 = (_NBC - 2 + par) * _BC
        for cond, src, dst, sem in store_descs(par, b0):
            @pl.when(cond)
            def _w(src=src, dst=dst, sem=sem):
                pltpu.make_async_copy(src, dst, sem).wait()


def kernel(idx, embedding):
    tab = jnp.pad(embedding.T, ((0, 0), (0, _VPAD - _VOCAB)))
    return _tgather_kernel(tab.reshape(-1), idx).T
